# baseline (speedup 1.0000x reference)
import functools

import jax
import jax.numpy as jnp
from jax import lax
from jax.experimental import pallas as pl
from jax.experimental.pallas import tpu as pltpu
from jax.experimental.pallas import tpu_sc as plsc

B = 16
N = 8192
C = 3
G = 512
K = 32
L = 16
NCHUNK = N // L
BIG_I = 1 << 30


def _iota():
    return lax.iota(jnp.int32, L)


def _splat_f(x):
    return jnp.full((L,), x, dtype=jnp.float32)


def _splat_i(x):
    return jnp.full((L,), x, dtype=jnp.int32)


def _round_bf16(v):
    u = plsc.bitcast(v, jnp.int32)
    r = (u + 0x7FFF + ((u >> 16) & 1)) & -65536
    return plsc.bitcast(r, jnp.float32)


def _permute(x, idx):
    dn = lax.GatherDimensionNumbers(
        offset_dims=(), collapsed_slice_dims=(0,), start_index_map=(0,)
    )
    return lax.gather(
        x, idx[:, None], dn, slice_sizes=(1,),
        mode=lax.GatherScatterMode.PROMISE_IN_BOUNDS,
    )


def _lex_lt(ka, ia, kb, ib):
    return (ka < kb) | ((ka == kb) & (ia < ib))


def _merge16(ak, ai, bk, bi):
    revp = L - 1 - _iota()
    brk = _permute(bk, revp)
    bri = _permute(bi, revp)
    sel = _lex_lt(ak, ai, brk, bri)
    lok = jnp.where(sel, ak, brk)
    loi = jnp.where(sel, ai, bri)
    hik = jnp.where(sel, brk, ak)
    hii = jnp.where(sel, bri, ai)

    def clean(k, i):
        for s in (8, 4, 2, 1):
            p = _iota() ^ s
            upper = (_iota() & s) != 0
            pk = _permute(k, p)
            pi = _permute(i, p)
            small = _lex_lt(k, i, pk, pi)
            keep = small ^ upper
            k = jnp.where(keep, k, pk)
            i = jnp.where(keep, i, pi)
        return k, i

    lok, loi = clean(lok, loi)
    hik, hii = clean(hik, hii)
    return lok, loi, hik, hii


def _top32_insert(carry, ck, ci):
    ak, ai, bk, bi = carry
    mlo_k, mlo_i, _, _ = _merge16(bk, bi, ck, ci)
    return _merge16(ak, ai, mlo_k, mlo_i)


def _fps_body(pts_hbm, ctr_hbm, xr, yr, zr, dr, cb):
    w = lax.axis_index("s") * 2 + lax.axis_index("c")

    @pl.when(w < B)
    def _():
        b = w
        pltpu.sync_copy(pts_hbm.at[pl.ds((b * 3 + 0) * N, N)], xr)
        pltpu.sync_copy(pts_hbm.at[pl.ds((b * 3 + 1) * N, N)], yr)
        pltpu.sync_copy(pts_hbm.at[pl.ds((b * 3 + 2) * N, N)], zr)

        def init_body(j, _):
            for u in range(8):
                dr[pl.ds((j * 8 + u) * L, L)] = _splat_f(1e10)
            return 0

        lax.fori_loop(0, NCHUNK // 8, init_body, 0)

        iota = _iota()
        lane0 = iota == 0
        lane1 = iota == 1
        mask3 = iota < 3

        def step(s, fvec):
            cx = plsc.load_gather(xr, [fvec])
            cy = plsc.load_gather(yr, [fvec])
            cz = plsc.load_gather(zr, [fvec])
            v3 = jnp.where(lane0, cx, jnp.where(lane1, cy, cz))
            plsc.store_scatter(cb, [_splat_i(3 * s) + iota], v3, mask=mask3)

            def chunk(j, st):
                st = list(st)
                for u in range(4):
                    bestv, besti = st[2 * u], st[2 * u + 1]
                    ds_ = pl.ds((j * 4 + u) * L, L)
                    dx = xr[ds_] - cx
                    dy = yr[ds_] - cy
                    dz = zr[ds_] - cz
                    d = (dx * dx + dy * dy) + dz * dz
                    dm = jnp.minimum(dr[ds_], d)
                    dr[ds_] = dm
                    upd = dm > bestv
                    st[2 * u] = jnp.where(upd, dm, bestv)
                    st[2 * u + 1] = jnp.where(
                        upd, _splat_i((j * 4 + u) * L) + iota, besti)
                return tuple(st)

            acc = lax.fori_loop(
                0, NCHUNK // 4, chunk,
                (_splat_f(-jnp.inf), _splat_i(0)) * 4,
            )
            bestv, besti = acc[0], acc[1]
            for u in range(1, 4):
                vu, iu = acc[2 * u], acc[2 * u + 1]
                take = (vu > bestv) | ((vu == bestv) & (iu < besti))
                bestv = jnp.where(take, vu, bestv)
                besti = jnp.where(take, iu, besti)
            vmax = jnp.max(bestv)
            cand = jnp.where(bestv == vmax, besti, BIG_I)
            nxt = jnp.min(cand)
            return _splat_i(nxt)

        lax.fori_loop(0, G, step, _splat_i(0))
        pltpu.sync_copy(cb.at[pl.ds(0, 3 * G)], ctr_hbm.at[pl.ds(b * 3 * G, 3 * G)])


_GPW = G // 2


def _knn_body(pts_hbm, ctr_hbm, out_hbm, xr, yr, zr, pnr, dbuf, ci_buf, cbv, ob,
              xb, yb, zb):
    w = lax.axis_index("s") * 2 + lax.axis_index("c")
    b = w // 2
    h = w % 2

    pltpu.sync_copy(pts_hbm.at[pl.ds((b * 3 + 0) * N, N)], xr)
    pltpu.sync_copy(pts_hbm.at[pl.ds((b * 3 + 1) * N, N)], yr)
    pltpu.sync_copy(pts_hbm.at[pl.ds((b * 3 + 2) * N, N)], zr)
    pltpu.sync_copy(ctr_hbm.at[pl.ds(b * 3 * G + h * 3 * _GPW, 3 * _GPW)], cbv)

    iota = _iota()
    inf = _splat_f(jnp.inf)

    def pn_body(j, _):
        for u in range(4):
            ds_ = pl.ds((j * 4 + u) * L, L)
            x = xr[ds_]
            y = yr[ds_]
            z = zr[ds_]
            pnr[ds_] = (x * x + y * y) + z * z
            xb[ds_] = _round_bf16(x)
            yb[ds_] = _round_bf16(y)
            zb[ds_] = _round_bf16(z)
        return 0

    lax.fori_loop(0, NCHUNK // 4, pn_body, 0)

    def row(r, _):
        cx = plsc.load_gather(cbv, [_splat_i(3 * r)])
        cy = plsc.load_gather(cbv, [_splat_i(3 * r + 1)])
        cz = plsc.load_gather(cbv, [_splat_i(3 * r + 2)])
        cn = (cx * cx + cy * cy) + cz * cz
        cxb = _round_bf16(cx)
        cyb = _round_bf16(cy)
        czb = _round_bf16(cz)

        def p1(j, st):
            st = list(st)
            for u in range(4):
                m1, m2 = st[2 * u], st[2 * u + 1]
                ds_ = pl.ds((j * 4 + u) * L, L)
                dot = (cxb * xb[ds_] + cyb * yb[ds_]) + czb * zb[ds_]
                d = (cn + pnr[ds_]) - 2.0 * dot
                dbuf[ds_] = d
                c1 = d < m1
                st[2 * u] = jnp.where(c1, d, m1)
                st[2 * u + 1] = jnp.minimum(m2, jnp.where(c1, m1, d))
            return tuple(st)

        acc = lax.fori_loop(0, NCHUNK // 4, p1, (inf, inf) * 4)
        m1, m2 = acc[0], acc[1]
        for u in range(1, 4):
            m1u, m2u = acc[2 * u], acc[2 * u + 1]
            m2 = jnp.minimum(jnp.maximum(m1, m1u), jnp.minimum(m2, m2u))
            m1 = jnp.minimum(m1, m1u)
        t = jnp.max(m2)

        def p2(j, off):
            for u in range(2):
                ds_ = pl.ds((j * 2 + u) * L, L)
                d = dbuf[ds_]
                msk = d <= t
                plsc.store_compressed(
                    ci_buf.at[pl.ds(off, L)],
                    _splat_i((j * 2 + u) * L) + iota, mask=msk)
                cntv = plsc.all_reduce_population_count(msk)
                off = off + cntv[0]
            return off

        off = lax.fori_loop(0, NCHUNK // 2, p2, jnp.int32(0))
        plsc.store_scatter(ci_buf, [_splat_i(off) + iota], iota)

        def p3(cchunk, carry):
            ci = ci_buf[pl.ds(cchunk * L, L)]
            cd = plsc.load_gather(dbuf, [ci])
            valid = (_splat_i(cchunk * L) + iota) < off
            cd = jnp.where(valid, cd, inf)
            sk, si = plsc.sort_key_val(cd, ci)
            return _top32_insert(carry, sk, si)

        nch = (off + L - 1) // L
        init = (inf, _splat_i(BIG_I), inf, _splat_i(BIG_I))
        ak, ai, bk, bi = lax.fori_loop(0, nch, p3, init)

        base = _splat_i(r * 3 * K)
        for kv, idxv, j0 in ((ak, ai, 0), (bk, bi, L)):
            del kv
            gx = plsc.load_gather(xr, [idxv]) - cx
            gy = plsc.load_gather(yr, [idxv]) - cy
            gz = plsc.load_gather(zr, [idxv]) - cz
            o = base + _splat_i(3 * j0) + 3 * iota
            plsc.store_scatter(ob, [o], gx)
            plsc.store_scatter(ob, [o + 1], gy)
            plsc.store_scatter(ob, [o + 2], gz)
        return 0

    lax.fori_loop(0, _GPW, row, 0)
    pltpu.sync_copy(ob, out_hbm.at[pl.ds(w * _GPW * 3 * K, _GPW * 3 * K)])


@functools.cache
def _build():
    mesh = plsc.VectorSubcoreMesh(core_axis_name="c", subcore_axis_name="s")
    params = pltpu.CompilerParams(needs_layout_passes=False)
    fps = functools.partial(
        pl.kernel,
        mesh=mesh,
        compiler_params=params,
        out_type=jax.ShapeDtypeStruct((B * 3 * G,), jnp.float32),
        scratch_types=[
            pltpu.VMEM((N,), jnp.float32),
            pltpu.VMEM((N,), jnp.float32),
            pltpu.VMEM((N,), jnp.float32),
            pltpu.VMEM((N,), jnp.float32),
            pltpu.VMEM((3 * G + 16,), jnp.float32),
        ],
    )(_fps_body)
    knn = functools.partial(
        pl.kernel,
        mesh=mesh,
        compiler_params=params,
        out_type=jax.ShapeDtypeStruct((B * 2 * _GPW * 3 * K,), jnp.float32),
        scratch_types=[
            pltpu.VMEM((N,), jnp.float32),
            pltpu.VMEM((N,), jnp.float32),
            pltpu.VMEM((N,), jnp.float32),
            pltpu.VMEM((N,), jnp.float32),
            pltpu.VMEM((N,), jnp.float32),
            pltpu.VMEM((N + 2 * L,), jnp.int32),
            pltpu.VMEM((3 * _GPW,), jnp.float32),
            pltpu.VMEM((_GPW * 3 * K,), jnp.float32),
            pltpu.VMEM((N,), jnp.float32),
            pltpu.VMEM((N,), jnp.float32),
            pltpu.VMEM((N,), jnp.float32),
        ],
    )(_knn_body)
    return fps, knn


def kernel(points):
    fps_sc, knn_sc = _build()
    pts_t = jnp.transpose(points, (0, 2, 1)).reshape(-1)
    ctr = fps_sc(pts_t)
    nbr = knn_sc(pts_t, ctr)
    center = ctr.reshape(B, G, 3)
    neighborhood = nbr.reshape(B, G, K, 3)
    return (neighborhood, center)


# speedup vs baseline: 1.6807x; 1.6807x over previous
import functools

import jax
import jax.numpy as jnp
from jax import lax
from jax.experimental import pallas as pl
from jax.experimental.pallas import tpu as pltpu
from jax.experimental.pallas import tpu_sc as plsc

B = 16
N = 8192
C = 3
G = 512
K = 32
L = 16
NCHUNK = N // L
BIG_I = 1 << 30


def _iota():
    return lax.iota(jnp.int32, L)


def _splat_f(x):
    return jnp.full((L,), x, dtype=jnp.float32)


def _splat_i(x):
    return jnp.full((L,), x, dtype=jnp.int32)


def _round_bf16(v):
    u = plsc.bitcast(v, jnp.int32)
    r = (u + 0x7FFF + ((u >> 16) & 1)) & -65536
    return plsc.bitcast(r, jnp.float32)


def _permute(x, idx):
    dn = lax.GatherDimensionNumbers(
        offset_dims=(), collapsed_slice_dims=(0,), start_index_map=(0,)
    )
    return lax.gather(
        x, idx[:, None], dn, slice_sizes=(1,),
        mode=lax.GatherScatterMode.PROMISE_IN_BOUNDS,
    )


def _lex_lt(ka, ia, kb, ib):
    return (ka < kb) | ((ka == kb) & (ia < ib))


def _merge16(ak, ai, bk, bi):
    revp = L - 1 - _iota()
    brk = _permute(bk, revp)
    bri = _permute(bi, revp)
    sel = _lex_lt(ak, ai, brk, bri)
    lok = jnp.where(sel, ak, brk)
    loi = jnp.where(sel, ai, bri)
    hik = jnp.where(sel, brk, ak)
    hii = jnp.where(sel, bri, ai)

    def clean(k, i):
        for s in (8, 4, 2, 1):
            p = _iota() ^ s
            upper = (_iota() & s) != 0
            pk = _permute(k, p)
            pi = _permute(i, p)
            small = _lex_lt(k, i, pk, pi)
            keep = small ^ upper
            k = jnp.where(keep, k, pk)
            i = jnp.where(keep, i, pi)
        return k, i

    lok, loi = clean(lok, loi)
    hik, hii = clean(hik, hii)
    return lok, loi, hik, hii


def _top32_insert(carry, ck, ci):
    ak, ai, bk, bi = carry
    mlo_k, mlo_i, _, _ = _merge16(bk, bi, ck, ci)
    return _merge16(ak, ai, mlo_k, mlo_i)


def _fps_body(pts_hbm, ctr_hbm, xr, yr, zr, dr, cb):
    w = lax.axis_index("s") * 2 + lax.axis_index("c")

    @pl.when(w < B)
    def _():
        b = w
        pltpu.sync_copy(pts_hbm.at[pl.ds((b * 3 + 0) * N, N)], xr)
        pltpu.sync_copy(pts_hbm.at[pl.ds((b * 3 + 1) * N, N)], yr)
        pltpu.sync_copy(pts_hbm.at[pl.ds((b * 3 + 2) * N, N)], zr)

        @plsc.parallel_loop(0, NCHUNK, unroll=8)
        def _init(j):
            dr[pl.ds(j * L, L)] = _splat_f(1e10)

        iota = _iota()
        lane0 = iota == 0
        lane1 = iota == 1
        mask3 = iota < 3

        def step(s, fvec):
            cx = plsc.load_gather(xr, [fvec])
            cy = plsc.load_gather(yr, [fvec])
            cz = plsc.load_gather(zr, [fvec])
            v3 = jnp.where(lane0, cx, jnp.where(lane1, cy, cz))
            plsc.store_scatter(cb, [_splat_i(3 * s) + iota], v3, mask=mask3)

            def chunk(j, st):
                bestv, besti = st
                ds_ = pl.ds(j * L, L)
                dx = xr[ds_] - cx
                dy = yr[ds_] - cy
                dz = zr[ds_] - cz
                d = (dx * dx + dy * dy) + dz * dz
                dm = jnp.minimum(dr[ds_], d)
                dr[ds_] = dm
                upd = dm > bestv
                bestv = jnp.where(upd, dm, bestv)
                besti = jnp.where(upd, _splat_i(j * L) + iota, besti)
                return bestv, besti

            bestv, besti = plsc.parallel_loop(
                0, NCHUNK, unroll=4,
                carry=(_splat_f(-jnp.inf), _splat_i(0)),
            )(chunk)
            vmax = jnp.max(bestv)
            cand = jnp.where(bestv == vmax, besti, BIG_I)
            nxt = jnp.min(cand)
            return _splat_i(nxt)

        lax.fori_loop(0, G, step, _splat_i(0))
        pltpu.sync_copy(cb.at[pl.ds(0, 3 * G)], ctr_hbm.at[pl.ds(b * 3 * G, 3 * G)])


_GPW = G // 2


def _knn_body(pts_hbm, ctr_hbm, out_hbm, xr, yr, zr, pnr, dbuf, ci_buf, cbv, ob,
              xb, yb, zb):
    w = lax.axis_index("s") * 2 + lax.axis_index("c")
    b = w // 2
    h = w % 2

    pltpu.sync_copy(pts_hbm.at[pl.ds((b * 3 + 0) * N, N)], xr)
    pltpu.sync_copy(pts_hbm.at[pl.ds((b * 3 + 1) * N, N)], yr)
    pltpu.sync_copy(pts_hbm.at[pl.ds((b * 3 + 2) * N, N)], zr)
    pltpu.sync_copy(ctr_hbm.at[pl.ds(b * 3 * G + h * 3 * _GPW, 3 * _GPW)], cbv)

    iota = _iota()
    inf = _splat_f(jnp.inf)

    @plsc.parallel_loop(0, NCHUNK, unroll=4)
    def _pn(j):
        ds_ = pl.ds(j * L, L)
        x = xr[ds_]
        y = yr[ds_]
        z = zr[ds_]
        pnr[ds_] = (x * x + y * y) + z * z
        xb[ds_] = _round_bf16(x)
        yb[ds_] = _round_bf16(y)
        zb[ds_] = _round_bf16(z)

    def row(r, _):
        cx = plsc.load_gather(cbv, [_splat_i(3 * r)])
        cy = plsc.load_gather(cbv, [_splat_i(3 * r + 1)])
        cz = plsc.load_gather(cbv, [_splat_i(3 * r + 2)])
        cn = (cx * cx + cy * cy) + cz * cz
        cxb = _round_bf16(cx)
        cyb = _round_bf16(cy)
        czb = _round_bf16(cz)

        def p1(j, st):
            m1, m2 = st
            ds_ = pl.ds(j * L, L)
            dot = (cxb * xb[ds_] + cyb * yb[ds_]) + czb * zb[ds_]
            d = (cn + pnr[ds_]) - 2.0 * dot
            dbuf[ds_] = d
            c1 = d < m1
            m1n = jnp.where(c1, d, m1)
            m2n = jnp.minimum(m2, jnp.where(c1, m1, d))
            return m1n, m2n

        _, m2 = plsc.parallel_loop(0, NCHUNK, unroll=4, carry=(inf, inf))(p1)
        t = jnp.max(m2)

        def p2(j, off):
            for u in range(2):
                ds_ = pl.ds((j * 2 + u) * L, L)
                d = dbuf[ds_]
                msk = d <= t
                plsc.store_compressed(
                    ci_buf.at[pl.ds(off, L)],
                    _splat_i((j * 2 + u) * L) + iota, mask=msk)
                cntv = plsc.all_reduce_population_count(msk)
                off = off + cntv[0]
            return off

        off = lax.fori_loop(0, NCHUNK // 2, p2, jnp.int32(0))
        plsc.store_scatter(ci_buf, [_splat_i(off) + iota], iota)

        def p3(cchunk, carry):
            ci = ci_buf[pl.ds(cchunk * L, L)]
            cd = plsc.load_gather(dbuf, [ci])
            valid = (_splat_i(cchunk * L) + iota) < off
            cd = jnp.where(valid, cd, inf)
            sk, si = plsc.sort_key_val(cd, ci)
            return _top32_insert(carry, sk, si)

        nch = (off + L - 1) // L
        init = (inf, _splat_i(BIG_I), inf, _splat_i(BIG_I))
        ak, ai, bk, bi = lax.fori_loop(0, nch, p3, init)

        base = _splat_i(r * 3 * K)
        for kv, idxv, j0 in ((ak, ai, 0), (bk, bi, L)):
            del kv
            gx = plsc.load_gather(xr, [idxv]) - cx
            gy = plsc.load_gather(yr, [idxv]) - cy
            gz = plsc.load_gather(zr, [idxv]) - cz
            o = base + _splat_i(3 * j0) + 3 * iota
            plsc.store_scatter(ob, [o], gx)
            plsc.store_scatter(ob, [o + 1], gy)
            plsc.store_scatter(ob, [o + 2], gz)
        return 0

    lax.fori_loop(0, _GPW, row, 0)
    pltpu.sync_copy(ob, out_hbm.at[pl.ds(w * _GPW * 3 * K, _GPW * 3 * K)])


@functools.cache
def _build():
    mesh = plsc.VectorSubcoreMesh(core_axis_name="c", subcore_axis_name="s")
    params = pltpu.CompilerParams(needs_layout_passes=False)
    fps = functools.partial(
        pl.kernel,
        mesh=mesh,
        compiler_params=params,
        out_type=jax.ShapeDtypeStruct((B * 3 * G,), jnp.float32),
        scratch_types=[
            pltpu.VMEM((N,), jnp.float32),
            pltpu.VMEM((N,), jnp.float32),
            pltpu.VMEM((N,), jnp.float32),
            pltpu.VMEM((N,), jnp.float32),
            pltpu.VMEM((3 * G + 16,), jnp.float32),
        ],
    )(_fps_body)
    knn = functools.partial(
        pl.kernel,
        mesh=mesh,
        compiler_params=params,
        out_type=jax.ShapeDtypeStruct((B * 2 * _GPW * 3 * K,), jnp.float32),
        scratch_types=[
            pltpu.VMEM((N,), jnp.float32),
            pltpu.VMEM((N,), jnp.float32),
            pltpu.VMEM((N,), jnp.float32),
            pltpu.VMEM((N,), jnp.float32),
            pltpu.VMEM((N,), jnp.float32),
            pltpu.VMEM((N + 2 * L,), jnp.int32),
            pltpu.VMEM((3 * _GPW,), jnp.float32),
            pltpu.VMEM((_GPW * 3 * K,), jnp.float32),
            pltpu.VMEM((N,), jnp.float32),
            pltpu.VMEM((N,), jnp.float32),
            pltpu.VMEM((N,), jnp.float32),
        ],
    )(_knn_body)
    return fps, knn


def kernel(points):
    fps_sc, knn_sc = _build()
    pts_t = jnp.transpose(points, (0, 2, 1)).reshape(-1)
    ctr = fps_sc(pts_t)
    nbr = knn_sc(pts_t, ctr)
    center = ctr.reshape(B, G, 3)
    neighborhood = nbr.reshape(B, G, K, 3)
    return (neighborhood, center)


# speedup vs baseline: 2.9128x; 1.7331x over previous
import functools

import jax
import jax.numpy as jnp
from jax import lax
from jax.experimental import pallas as pl
from jax.experimental.pallas import tpu as pltpu
from jax.experimental.pallas import tpu_sc as plsc

B = 16
N = 8192
C = 3
G = 512
K = 32
L = 16
NCHUNK = N // L
BIG_I = 1 << 30


def _iota():
    return lax.iota(jnp.int32, L)


def _splat_f(x):
    return jnp.full((L,), x, dtype=jnp.float32)


def _splat_i(x):
    return jnp.full((L,), x, dtype=jnp.int32)


def _round_bf16(v):
    u = plsc.bitcast(v, jnp.int32)
    r = (u + 0x7FFF + ((u >> 16) & 1)) & -65536
    return plsc.bitcast(r, jnp.float32)


def _permute(x, idx):
    dn = lax.GatherDimensionNumbers(
        offset_dims=(), collapsed_slice_dims=(0,), start_index_map=(0,)
    )
    return lax.gather(
        x, idx[:, None], dn, slice_sizes=(1,),
        mode=lax.GatherScatterMode.PROMISE_IN_BOUNDS,
    )


def _lex_lt(ka, ia, kb, ib):
    return (ka < kb) | ((ka == kb) & (ia < ib))


def _merge16(ak, ai, bk, bi):
    revp = L - 1 - _iota()
    brk = _permute(bk, revp)
    bri = _permute(bi, revp)
    sel = _lex_lt(ak, ai, brk, bri)
    lok = jnp.where(sel, ak, brk)
    loi = jnp.where(sel, ai, bri)
    hik = jnp.where(sel, brk, ak)
    hii = jnp.where(sel, bri, ai)

    def clean(k, i):
        for s in (8, 4, 2, 1):
            p = _iota() ^ s
            upper = (_iota() & s) != 0
            pk = _permute(k, p)
            pi = _permute(i, p)
            small = _lex_lt(k, i, pk, pi)
            keep = small ^ upper
            k = jnp.where(keep, k, pk)
            i = jnp.where(keep, i, pi)
        return k, i

    lok, loi = clean(lok, loi)
    hik, hii = clean(hik, hii)
    return lok, loi, hik, hii


def _top32_insert(carry, ck, ci):
    ak, ai, bk, bi = carry
    mlo_k, mlo_i, _, _ = _merge16(bk, bi, ck, ci)
    return _merge16(ak, ai, mlo_k, mlo_i)


def _fps_body(pts_hbm, ctr_hbm, xr, yr, zr, dr, cb):
    w = lax.axis_index("s") * 2 + lax.axis_index("c")

    @pl.when(w < B)
    def _():
        b = w
        pltpu.sync_copy(pts_hbm.at[pl.ds((b * 3 + 0) * N, N)], xr)
        pltpu.sync_copy(pts_hbm.at[pl.ds((b * 3 + 1) * N, N)], yr)
        pltpu.sync_copy(pts_hbm.at[pl.ds((b * 3 + 2) * N, N)], zr)

        @plsc.parallel_loop(0, NCHUNK, unroll=8)
        def _init(j):
            dr[pl.ds(j * L, L)] = _splat_f(1e10)

        iota = _iota()
        lane0 = iota == 0
        lane1 = iota == 1
        mask3 = iota < 3

        def step(s, fvec):
            cx = plsc.load_gather(xr, [fvec])
            cy = plsc.load_gather(yr, [fvec])
            cz = plsc.load_gather(zr, [fvec])
            v3 = jnp.where(lane0, cx, jnp.where(lane1, cy, cz))
            plsc.store_scatter(cb, [_splat_i(3 * s) + iota], v3, mask=mask3)

            def chunk(j, st):
                bestv, besti = st
                ds_ = pl.ds(j * L, L)
                dx = xr[ds_] - cx
                dy = yr[ds_] - cy
                dz = zr[ds_] - cz
                d = (dx * dx + dy * dy) + dz * dz
                dm = jnp.minimum(dr[ds_], d)
                dr[ds_] = dm
                upd = dm > bestv
                bestv = jnp.where(upd, dm, bestv)
                besti = jnp.where(upd, _splat_i(j * L) + iota, besti)
                return bestv, besti

            bestv, besti = plsc.parallel_loop(
                0, NCHUNK, unroll=4,
                carry=(_splat_f(-jnp.inf), _splat_i(0)),
            )(chunk)
            vmax = jnp.max(bestv)
            cand = jnp.where(bestv == vmax, besti, BIG_I)
            nxt = jnp.min(cand)
            return _splat_i(nxt)

        lax.fori_loop(0, G, step, _splat_i(0))
        pltpu.sync_copy(cb.at[pl.ds(0, 3 * G)], ctr_hbm.at[pl.ds(b * 3 * G, 3 * G)])


_GPW = G // 2


def _knn_body(pts_hbm, ctr_hbm, out_hbm, xr, yr, zr, pnr, dbuf, ci_buf, cbv, ob,
              xb, yb, zb):
    w = lax.axis_index("s") * 2 + lax.axis_index("c")
    b = w // 2
    h = w % 2

    pltpu.sync_copy(pts_hbm.at[pl.ds((b * 3 + 0) * N, N)], xr)
    pltpu.sync_copy(pts_hbm.at[pl.ds((b * 3 + 1) * N, N)], yr)
    pltpu.sync_copy(pts_hbm.at[pl.ds((b * 3 + 2) * N, N)], zr)
    pltpu.sync_copy(ctr_hbm.at[pl.ds(b * 3 * G + h * 3 * _GPW, 3 * _GPW)], cbv)

    iota = _iota()
    inf = _splat_f(jnp.inf)

    @plsc.parallel_loop(0, NCHUNK, unroll=4)
    def _pn(j):
        ds_ = pl.ds(j * L, L)
        x = xr[ds_]
        y = yr[ds_]
        z = zr[ds_]
        pnr[ds_] = (x * x + y * y) + z * z
        xb[ds_] = _round_bf16(x)
        yb[ds_] = _round_bf16(y)
        zb[ds_] = _round_bf16(z)

    def row(r, _):
        cx = plsc.load_gather(cbv, [_splat_i(3 * r)])
        cy = plsc.load_gather(cbv, [_splat_i(3 * r + 1)])
        cz = plsc.load_gather(cbv, [_splat_i(3 * r + 2)])
        cn = (cx * cx + cy * cy) + cz * cz
        cxb = _round_bf16(cx)
        cyb = _round_bf16(cy)
        czb = _round_bf16(cz)

        def p1(j, st):
            m1, m2 = st
            ds_ = pl.ds(j * L, L)
            dot = (cxb * xb[ds_] + cyb * yb[ds_]) + czb * zb[ds_]
            d = (cn + pnr[ds_]) - 2.0 * dot
            dbuf[ds_] = d
            c1 = d < m1
            m1n = jnp.where(c1, d, m1)
            m2n = jnp.minimum(m2, jnp.where(c1, m1, d))
            return m1n, m2n

        _, m2 = plsc.parallel_loop(0, NCHUNK, unroll=4, carry=(inf, inf))(p1)
        t = jnp.max(m2)

        def p2(j, off):
            ds_ = pl.ds(j * L, L)
            d = dbuf[ds_]
            msk = d <= t
            plsc.store_compressed(
                ci_buf.at[pl.ds(off, L)], _splat_i(j * L) + iota, mask=msk)
            cntv = plsc.all_reduce_population_count(msk)
            return off + cntv[0]

        off = plsc.parallel_loop(
            0, NCHUNK, unroll=4, carry=jnp.int32(0))(p2)
        plsc.store_scatter(ci_buf, [_splat_i(off) + iota], iota)

        def p3(cchunk, carry):
            ci = ci_buf[pl.ds(cchunk * L, L)]
            cd = plsc.load_gather(dbuf, [ci])
            valid = (_splat_i(cchunk * L) + iota) < off
            cd = jnp.where(valid, cd, inf)
            sk, si = plsc.sort_key_val(cd, ci)
            return _top32_insert(carry, sk, si)

        nch = (off + L - 1) // L
        init = (inf, _splat_i(BIG_I), inf, _splat_i(BIG_I))
        ak, ai, bk, bi = plsc.parallel_loop(
            0, nch, unroll=2, carry=init)(lambda j, c: p3(j, c))

        base = _splat_i(r * 3 * K)
        for kv, idxv, j0 in ((ak, ai, 0), (bk, bi, L)):
            del kv
            gx = plsc.load_gather(xr, [idxv]) - cx
            gy = plsc.load_gather(yr, [idxv]) - cy
            gz = plsc.load_gather(zr, [idxv]) - cz
            o = base + _splat_i(3 * j0) + 3 * iota
            plsc.store_scatter(ob, [o], gx)
            plsc.store_scatter(ob, [o + 1], gy)
            plsc.store_scatter(ob, [o + 2], gz)
        return 0

    lax.fori_loop(0, _GPW, row, 0)
    pltpu.sync_copy(ob, out_hbm.at[pl.ds(w * _GPW * 3 * K, _GPW * 3 * K)])


@functools.cache
def _build():
    mesh = plsc.VectorSubcoreMesh(core_axis_name="c", subcore_axis_name="s")
    params = pltpu.CompilerParams(needs_layout_passes=False)
    fps = functools.partial(
        pl.kernel,
        mesh=mesh,
        compiler_params=params,
        out_type=jax.ShapeDtypeStruct((B * 3 * G,), jnp.float32),
        scratch_types=[
            pltpu.VMEM((N,), jnp.float32),
            pltpu.VMEM((N,), jnp.float32),
            pltpu.VMEM((N,), jnp.float32),
            pltpu.VMEM((N,), jnp.float32),
            pltpu.VMEM((3 * G + 16,), jnp.float32),
        ],
    )(_fps_body)
    knn = functools.partial(
        pl.kernel,
        mesh=mesh,
        compiler_params=params,
        out_type=jax.ShapeDtypeStruct((B * 2 * _GPW * 3 * K,), jnp.float32),
        scratch_types=[
            pltpu.VMEM((N,), jnp.float32),
            pltpu.VMEM((N,), jnp.float32),
            pltpu.VMEM((N,), jnp.float32),
            pltpu.VMEM((N,), jnp.float32),
            pltpu.VMEM((N,), jnp.float32),
            pltpu.VMEM((N + 2 * L,), jnp.int32),
            pltpu.VMEM((3 * _GPW,), jnp.float32),
            pltpu.VMEM((_GPW * 3 * K,), jnp.float32),
            pltpu.VMEM((N,), jnp.float32),
            pltpu.VMEM((N,), jnp.float32),
            pltpu.VMEM((N,), jnp.float32),
        ],
    )(_knn_body)
    return fps, knn


def kernel(points):
    fps_sc, knn_sc = _build()
    pts_t = jnp.transpose(points, (0, 2, 1)).reshape(-1)
    ctr = fps_sc(pts_t)
    nbr = knn_sc(pts_t, ctr)
    center = ctr.reshape(B, G, 3)
    neighborhood = nbr.reshape(B, G, K, 3)
    return (neighborhood, center)
